# baseline (speedup 1.0000x reference)
import functools

import jax
import jax.numpy as jnp
from jax import lax
from jax.experimental import pallas as pl
from jax.experimental.pallas import tpu as pltpu
from jax.experimental.pallas import tpu_sc as plsc

NUM_LOC = 128
B = 4096
NC = 2
NS = 16
L = 16
NC_USED = 1
NW = NC_USED * NS
CHUNK = B // NW

_mesh = plsc.VectorSubcoreMesh(core_axis_name="c", subcore_axis_name="s",
                               num_cores=1)


@functools.partial(
    pl.kernel,
    out_type=(
        jax.ShapeDtypeStruct((B,), jnp.float32),
        jax.ShapeDtypeStruct((B,), jnp.float32),
        jax.ShapeDtypeStruct((B,), jnp.float32),
    ),
    mesh=_mesh,
    compiler_params=pltpu.CompilerParams(needs_layout_passes=False),
    scratch_types=[
        pltpu.VMEM((CHUNK,), jnp.int32),
        pltpu.VMEM((NUM_LOC,), jnp.float32),
        pltpu.VMEM((NUM_LOC,), jnp.float32),
        pltpu.VMEM((NUM_LOC,), jnp.float32),
        pltpu.VMEM((CHUNK,), jnp.float32),
        pltpu.VMEM((CHUNK,), jnp.float32),
        pltpu.VMEM((CHUNK,), jnp.float32),
        pltpu.SemaphoreType.DMA,
    ],
)
def _lookup(loc_hbm, thc_hbm, thg_hbm, tbg_hbm,
            o1_hbm, o2_hbm, o3_hbm,
            idx_v, t1_v, t2_v, t3_v, o1_v, o2_v, o3_v, sem):
    wid = lax.axis_index("s") * NC_USED + lax.axis_index("c")
    base = wid * CHUNK
    c0 = pltpu.async_copy(loc_hbm.at[pl.ds(base, CHUNK)], idx_v, sem)
    c1 = pltpu.async_copy(thc_hbm, t1_v, sem)
    c2 = pltpu.async_copy(thg_hbm, t2_v, sem)
    c3 = pltpu.async_copy(tbg_hbm, t3_v, sem)
    c0.wait()
    c1.wait()
    c2.wait()
    c3.wait()
    for i in range(CHUNK // L):
        sl = pl.ds(i * L, L)
        o1_v[sl] = jnp.maximum(plsc.load_gather(t1_v, [idx_v[sl]]), 0.0)
    d1 = pltpu.async_copy(o1_v, o1_hbm.at[pl.ds(base, CHUNK)], sem)
    for i in range(CHUNK // L):
        sl = pl.ds(i * L, L)
        o2_v[sl] = jnp.maximum(plsc.load_gather(t2_v, [idx_v[sl]]), 0.0)
    d2 = pltpu.async_copy(o2_v, o2_hbm.at[pl.ds(base, CHUNK)], sem)
    for i in range(CHUNK // L):
        sl = pl.ds(i * L, L)
        o3_v[sl] = jnp.maximum(plsc.load_gather(t3_v, [idx_v[sl]]), 0.0) * 20.0
    d3 = pltpu.async_copy(o3_v, o3_hbm.at[pl.ds(base, CHUNK)], sem)
    d1.wait()
    d2.wait()
    d3.wait()


def kernel(location, th_c, th_g, tb_g):
    o1, o2, o3 = _lookup(location.astype(jnp.int32), th_c, th_g, tb_g)
    return (o1.reshape(-1, 1), o2.reshape(-1, 1), o3.reshape(-1, 1))


# speedup vs baseline: 1.0168x; 1.0168x over previous
import functools

import jax
import jax.numpy as jnp
from jax import lax
from jax.experimental import pallas as pl
from jax.experimental.pallas import tpu as pltpu
from jax.experimental.pallas import tpu_sc as plsc

NUM_LOC = 128
B = 4096
NC = 2
NS = 16
L = 16
NC_USED = 1
NW = NC_USED * NS
CHUNK = B // NW

_mesh = plsc.VectorSubcoreMesh(core_axis_name="c", subcore_axis_name="s",
                               num_cores=1)


@functools.partial(
    pl.kernel,
    out_type=(
        jax.ShapeDtypeStruct((B,), jnp.float32),
        jax.ShapeDtypeStruct((B,), jnp.float32),
        jax.ShapeDtypeStruct((B,), jnp.float32),
    ),
    mesh=_mesh,
    compiler_params=pltpu.CompilerParams(needs_layout_passes=False),
    scratch_types=[
        pltpu.VMEM((CHUNK,), jnp.int32),
        pltpu.VMEM((NUM_LOC,), jnp.float32),
        pltpu.VMEM((NUM_LOC,), jnp.float32),
        pltpu.VMEM((NUM_LOC,), jnp.float32),
        pltpu.VMEM((CHUNK,), jnp.float32),
        pltpu.VMEM((CHUNK,), jnp.float32),
        pltpu.VMEM((CHUNK,), jnp.float32),
        pltpu.SemaphoreType.DMA,
    ],
)
def _lookup(loc_hbm, thc_hbm, thg_hbm, tbg_hbm,
            o1_hbm, o2_hbm, o3_hbm,
            idx_v, t1_v, t2_v, t3_v, o1_v, o2_v, o3_v, sem):
    wid = lax.axis_index("s") * NC_USED + lax.axis_index("c")
    base = wid * CHUNK
    c0 = pltpu.async_copy(loc_hbm.at[pl.ds(base, CHUNK)], idx_v, sem)
    c1 = pltpu.async_copy(thc_hbm, t1_v, sem)
    c2 = pltpu.async_copy(thg_hbm, t2_v, sem)
    c3 = pltpu.async_copy(tbg_hbm, t3_v, sem)
    c0.wait()
    c1.wait()
    c2.wait()
    c3.wait()
    for i in range(CHUNK // L):
        sl = pl.ds(i * L, L)
        idx = idx_v[sl]
        v1 = plsc.load_gather(t1_v, [idx])
        v2 = plsc.load_gather(t2_v, [idx])
        v3 = plsc.load_gather(t3_v, [idx])
        o1_v[sl] = jnp.maximum(v1, 0.0)
        o2_v[sl] = jnp.maximum(v2, 0.0)
        o3_v[sl] = jnp.maximum(v3, 0.0) * 20.0
    d1 = pltpu.async_copy(o1_v, o1_hbm.at[pl.ds(base, CHUNK)], sem)
    d2 = pltpu.async_copy(o2_v, o2_hbm.at[pl.ds(base, CHUNK)], sem)
    d3 = pltpu.async_copy(o3_v, o3_hbm.at[pl.ds(base, CHUNK)], sem)
    d1.wait()
    d2.wait()
    d3.wait()


def kernel(location, th_c, th_g, tb_g):
    o1, o2, o3 = _lookup(location.astype(jnp.int32), th_c, th_g, tb_g)
    return (o1.reshape(-1, 1), o2.reshape(-1, 1), o3.reshape(-1, 1))
